# baseline (speedup 1.0000x reference)
import jax
import jax.numpy as jnp
from jax import lax
from jax.experimental import pallas as pl
from jax.experimental.pallas import tpu as pltpu
from jax.experimental.pallas import tpu_sc as plsc

_TOT = 320
_B, _L = 4096, 50
_N = _B * _L
_CB = 128
_NC, _NS = 2, 16
_NW = _NC * _NS


def _sc_body(ic, ib, if_, ir, ip, wc, wb, wf, wr, wp, out,
             ivc, ivb, ivf, ivr, ivp, ivr2, ivp2, tb, tf,
             rc, rr, rp, xc, xb, xf, xr, xp, semg, semw):
    wid = lax.axis_index("s") * _NC + lax.axis_index("c")
    b0 = wid * _CB
    iota = lax.iota(jnp.int32, 16)

    pltpu.sync_copy(wb, tb)
    pltpu.sync_copy(wf, tf)

    def stage_idx(l):
        off = l * _B + b0
        for src, dst in ((ic, ivc), (ib, ivb), (if_, ivf),
                         (ir, ivr), (ip, ivp)):
            pltpu.sync_copy(src.at[pl.ds(off, _CB)], dst)
        for g in range(8):
            s = pl.ds(g * 16, 16)
            ivr2[s] = lax.shift_right_logical(ivr[s], 1)
            ivp2[s] = lax.shift_right_logical(ivp[s], 1)

    def fire_gathers():
        pltpu.make_async_copy(wc.at[ivc], rc, semg).start()
        pltpu.make_async_copy(wr.at[ivr2], rr, semg).start()
        pltpu.make_async_copy(wp.at[ivp2], rp, semg).start()

    def wait_gathers():
        for dst in (rc, rr, rp):
            pltpu.make_async_copy(out.at[pl.ds(0, _CB), pl.ds(0, 128)],
                                  dst, semg).wait()

    def fire_writes(l):
        r0 = l * _TOT
        for x, off, f in ((xc, 0, 128), (xb, 128, 32), (xf, 160, 32),
                          (xr, 192, 64), (xp, 256, 64)):
            pltpu.make_async_copy(
                x, out.at[pl.ds(r0 + off, f), pl.ds(b0, _CB)], semw).start()

    def wait_writes():
        for x, f in ((xc, 128), (xb, 32), (xf, 32), (xr, 64), (xp, 64)):
            pltpu.make_async_copy(
                x, out.at[pl.ds(0, f), pl.ds(0, _CB)], semw).wait()

    def assemble():
        def body(f2, carry):
            f8 = f2 * 8

            for g in range(8):
                rows = g * 16 + iota
                for u in range(8):
                    xc[f8 + u, pl.ds(g * 16, 16)] = plsc.load_gather(
                        rc, [rows, jnp.full((16,), f8 + u, jnp.int32)])

            @pl.when(f2 < 4)
            def _():
                for g in range(8):
                    s = pl.ds(g * 16, 16)
                    vb = ivb[s] * 32
                    vf = ivf[s] * 32
                    for u in range(8):
                        ff = jnp.full((16,), f8 + u, jnp.int32)
                        xb[f8 + u, s] = plsc.load_gather(tb, [vb + ff])
                        xf[f8 + u, s] = plsc.load_gather(tf, [vf + ff])

            @pl.when(f2 < 8)
            def _():
                for g in range(8):
                    s = pl.ds(g * 16, 16)
                    rows = g * 16 + iota
                    ro = (ivr[s] & 1) * 64
                    po = (ivp[s] & 1) * 64
                    for u in range(8):
                        ff = jnp.full((16,), f8 + u, jnp.int32)
                        xr[f8 + u, s] = plsc.load_gather(rr, [rows, ro + ff])
                        xp[f8 + u, s] = plsc.load_gather(rp, [rows, po + ff])
            return carry

        lax.fori_loop(0, 16, body, 0)

    stage_idx(0)
    fire_gathers()

    def step(l, carry):
        wait_gathers()

        @pl.when(l >= 1)
        def _():
            wait_writes()

        assemble()
        fire_writes(l)

        @pl.when(l < _L - 1)
        def _():
            stage_idx(l + 1)
            fire_gathers()

        return carry

    lax.fori_loop(0, _L, step, 0)
    wait_writes()


def kernel(idx_char, idx_bound, idx_flag, idx_radical, idx_pinyin,
           W_char, W_bound, W_flag, W_radical, W_pinyin):
    idxs = [a.T.reshape(_N).astype(jnp.int32)
            for a in (idx_char, idx_bound, idx_flag, idx_radical, idx_pinyin)]
    tables = [W_char,
              W_bound.reshape(100 * 32),
              W_flag.reshape(100 * 32),
              W_radical.reshape(500, 128),
              W_pinyin.reshape(500, 128)]
    scratch = ([pltpu.VMEM((_CB,), jnp.int32) for _ in range(7)]
               + [pltpu.VMEM((100 * 32,), jnp.float32) for _ in range(2)]
               + [pltpu.VMEM((_CB, 128), jnp.float32) for _ in range(3)]
               + [pltpu.VMEM((128, _CB), jnp.float32),
                  pltpu.VMEM((32, _CB), jnp.float32),
                  pltpu.VMEM((32, _CB), jnp.float32),
                  pltpu.VMEM((64, _CB), jnp.float32),
                  pltpu.VMEM((64, _CB), jnp.float32)]
               + [pltpu.SemaphoreType.DMA for _ in range(2)])
    k = pl.kernel(
        _sc_body,
        out_type=jax.ShapeDtypeStruct((_L * _TOT, _B), jnp.float32),
        mesh=plsc.VectorSubcoreMesh(core_axis_name="c", subcore_axis_name="s"),
        scratch_types=scratch,
        compiler_params=pltpu.CompilerParams(needs_layout_passes=False),
    )
    out = k(*idxs, *tables)
    return out.reshape(_L, _TOT, _B).transpose(2, 0, 1)


# speedup vs baseline: 1.4560x; 1.4560x over previous
import jax
import jax.numpy as jnp
from jax import lax
from jax.experimental import pallas as pl
from jax.experimental.pallas import tpu as pltpu
from jax.experimental.pallas import tpu_sc as plsc

_TOT = 320
_B, _L = 4096, 50
_N = _B * _L
_CB = 128
_NC, _NS = 2, 16
_NW = _NC * _NS


def _sc_body(ic, ib, if_, ir, ip, wc, wb, wf, wr, wp, out,
             ivc, ivb, ivf, ivr, ivp, ivr2, ivp2, tb, tf,
             rc, rr, rp, xc, xb, xf, xr, xp, semg, semw):
    wid = lax.axis_index("s") * _NC + lax.axis_index("c")
    b0 = wid * _CB
    iota = lax.iota(jnp.int32, 16)

    pltpu.sync_copy(wb, tb)
    pltpu.sync_copy(wf, tf)

    def stage_idx(l):
        off = l * _B + b0
        for src, dst in ((ic, ivc), (ib, ivb), (if_, ivf),
                         (ir, ivr), (ip, ivp)):
            pltpu.sync_copy(src.at[pl.ds(off, _CB)], dst)
        for g in range(8):
            s = pl.ds(g * 16, 16)
            ivr2[s] = lax.shift_right_logical(ivr[s], 1)
            ivp2[s] = lax.shift_right_logical(ivp[s], 1)

    def fire_gathers():
        pltpu.make_async_copy(wc.at[ivc], rc, semg).start()
        pltpu.make_async_copy(wr.at[ivr2], rr, semg).start()
        pltpu.make_async_copy(wp.at[ivp2], rp, semg).start()

    def wait_gathers():
        for dst in (rc, rr, rp):
            pltpu.make_async_copy(out.at[pl.ds(0, _CB), pl.ds(0, 128)],
                                  dst, semg).wait()

    def fire_writes(l):
        r0 = l * _TOT
        for x, off, f in ((xc, 0, 128), (xb, 128, 32), (xf, 160, 32),
                          (xr, 192, 64), (xp, 256, 64)):
            pltpu.make_async_copy(
                x, out.at[pl.ds(r0 + off, f), pl.ds(b0, _CB)], semw).start()

    def wait_writes():
        for x, f in ((xc, 128), (xb, 32), (xf, 32), (xr, 64), (xp, 64)):
            pltpu.make_async_copy(
                x, out.at[pl.ds(0, f), pl.ds(0, _CB)], semw).wait()

    def assemble():
        @plsc.parallel_loop(0, 16, 1, unroll=2)
        def _char(f2):
            f8 = f2 * 8
            for g in range(8):
                rows = g * 16 + iota
                for u in range(8):
                    xc[f8 + u, pl.ds(g * 16, 16)] = plsc.load_gather(
                        rc, [rows, jnp.full((16,), f8 + u, jnp.int32)])

        @plsc.parallel_loop(0, 4, 1, unroll=2)
        def _bf(f2):
            f8 = f2 * 8
            for g in range(8):
                s = pl.ds(g * 16, 16)
                vb = ivb[s] * 32
                vf = ivf[s] * 32
                for u in range(8):
                    ff = jnp.full((16,), f8 + u, jnp.int32)
                    xb[f8 + u, s] = plsc.load_gather(tb, [vb + ff])
                    xf[f8 + u, s] = plsc.load_gather(tf, [vf + ff])

        @plsc.parallel_loop(0, 8, 1, unroll=2)
        def _rp2(f2):
            f8 = f2 * 8
            for g in range(8):
                s = pl.ds(g * 16, 16)
                rows = g * 16 + iota
                ro = (ivr[s] & 1) * 64
                po = (ivp[s] & 1) * 64
                for u in range(8):
                    ff = jnp.full((16,), f8 + u, jnp.int32)
                    xr[f8 + u, s] = plsc.load_gather(rr, [rows, ro + ff])
                    xp[f8 + u, s] = plsc.load_gather(rp, [rows, po + ff])

    stage_idx(0)
    fire_gathers()

    def step(l, carry):
        wait_gathers()

        @pl.when(l >= 1)
        def _():
            wait_writes()

        assemble()
        fire_writes(l)

        @pl.when(l < _L - 1)
        def _():
            stage_idx(l + 1)
            fire_gathers()

        return carry

    lax.fori_loop(0, _L, step, 0)
    wait_writes()


def kernel(idx_char, idx_bound, idx_flag, idx_radical, idx_pinyin,
           W_char, W_bound, W_flag, W_radical, W_pinyin):
    idxs = [a.T.reshape(_N).astype(jnp.int32)
            for a in (idx_char, idx_bound, idx_flag, idx_radical, idx_pinyin)]
    tables = [W_char,
              W_bound.reshape(100 * 32),
              W_flag.reshape(100 * 32),
              W_radical.reshape(500, 128),
              W_pinyin.reshape(500, 128)]
    scratch = ([pltpu.VMEM((_CB,), jnp.int32) for _ in range(7)]
               + [pltpu.VMEM((100 * 32,), jnp.float32) for _ in range(2)]
               + [pltpu.VMEM((_CB, 128), jnp.float32) for _ in range(3)]
               + [pltpu.VMEM((128, _CB), jnp.float32),
                  pltpu.VMEM((32, _CB), jnp.float32),
                  pltpu.VMEM((32, _CB), jnp.float32),
                  pltpu.VMEM((64, _CB), jnp.float32),
                  pltpu.VMEM((64, _CB), jnp.float32)]
               + [pltpu.SemaphoreType.DMA for _ in range(2)])
    k = pl.kernel(
        _sc_body,
        out_type=jax.ShapeDtypeStruct((_L * _TOT, _B), jnp.float32),
        mesh=plsc.VectorSubcoreMesh(core_axis_name="c", subcore_axis_name="s"),
        scratch_types=scratch,
        compiler_params=pltpu.CompilerParams(needs_layout_passes=False),
    )
    out = k(*idxs, *tables)
    return out.reshape(_L, _TOT, _B).transpose(2, 0, 1)


# speedup vs baseline: 1.9599x; 1.3461x over previous
import functools

import jax
import jax.numpy as jnp
from jax import lax
from jax.experimental import pallas as pl
from jax.experimental.pallas import tpu as pltpu
from jax.experimental.pallas import tpu_sc as plsc

_TOT = 320
_B, _L = 4096, 50
_N = _B * _L
_C = 50
_NC, _NS = 2, 16
_NW = _NC * _NS
_BH = _B // 2
_NH = _BH * _L
_PER_W = _NH // _NW
_STEPS = _PER_W // _C


def _sc_body(half, ic, ib, if_, ir, ip, wc, wb, wf, wr, wp, out,
             ivc, ivb, ivf, ivr, ivp, tb, tf,
             q0c, q0r, q0p, q1c, q1r, q1p,
             c0, c1, r0, r1, p0, p1, a0, a1,
             sg0, sg1, sw0, sw1):
    wid = lax.axis_index("s") * _NC + lax.axis_index("c")
    tok0 = half * _NH + wid * _PER_W
    row_b0 = wid * _STEPS
    gsrc = (wc, wr, wp)
    qidx = ((q0c, q0r, q0p), (q1c, q1r, q1p))
    dsts = ((c0, r0, p0), (c1, r1, p1))
    asm = (a0, a1)
    semg = (sg0, sg1)
    semw = (sw0, sw1)

    for src, dst in ((ic, ivc), (ib, ivb), (if_, ivf), (ir, ivr), (ip, ivp)):
        pltpu.sync_copy(src.at[pl.ds(tok0, _PER_W)], dst)
    pltpu.sync_copy(wb, tb)
    pltpu.sync_copy(wf, tf)

    iota = lax.iota(jnp.int32, 16)
    tail_mask = iota < 2

    def repack(j, s):
        base = jnp.full((16,), j * _C, jnp.int32)
        for t, (flat, shift) in enumerate(((ivc, 0), (ivr, 1), (ivp, 1))):
            q = qidx[s][t]
            for c in range(3):
                v = plsc.load_gather(flat, [base + (c * 16 + iota)])
                q[pl.ds(c * 16, 16)] = lax.shift_right_logical(v, shift)
            v = plsc.load_gather(flat, [base + (48 + iota)])
            plsc.store_scatter(q, [48 + iota],
                               lax.shift_right_logical(v, shift),
                               mask=tail_mask)

    def fire_gathers(s):
        for t in range(3):
            pltpu.make_async_copy(
                gsrc[t].at[qidx[s][t]], dsts[s][t], semg[s]).start()

    def wait_gathers(s):
        for t in range(3):
            pltpu.make_async_copy(
                out.at[0, :, pl.ds(0, 128)], dsts[s][t], semg[s]).wait()

    def wait_writes(s):
        pltpu.make_async_copy(
            dsts[s][0], out.at[0, :, pl.ds(0, 128)], semw[s]).wait()
        pltpu.make_async_copy(
            asm[s], out.at[0, :, pl.ds(128, 192)], semw[s]).wait()

    repack(0, 0)
    fire_gathers(0)

    def step(j, carry):
        s = lax.rem(j, 2)

        @pl.when(s == 0)
        def _():
            wait_gathers(0)

        @pl.when(s == 1)
        def _():
            wait_gathers(1)

        @pl.when((j >= 1) & (s == 0))
        def _():
            wait_writes(1)

        @pl.when((j >= 1) & (s == 1))
        def _():
            wait_writes(0)

        @pl.when((j < _STEPS - 1) & (s == 0))
        def _():
            repack(j + 1, 1)
            fire_gathers(1)

        @pl.when((j < _STEPS - 1) & (s == 1))
        def _():
            repack(j + 1, 0)
            fire_gathers(0)

        def assemble(i, carry3, rr, rp, am):
            pos = jnp.full((16,), j * _C + i, jnp.int32)
            row = jnp.full((16,), i, jnp.int32)
            vb = plsc.load_gather(ivb, [pos]) * 32
            vf = plsc.load_gather(ivf, [pos]) * 32
            vr = plsc.load_gather(ivr, [pos])
            vp = plsc.load_gather(ivp, [pos])
            roff = (vr & 1) * 64
            poff = (vp & 1) * 64
            am[i, pl.ds(0, 16)] = plsc.load_gather(tb, [vb + iota])
            am[i, pl.ds(16, 16)] = plsc.load_gather(tb, [vb + (16 + iota)])
            am[i, pl.ds(32, 16)] = plsc.load_gather(tf, [vf + iota])
            am[i, pl.ds(48, 16)] = plsc.load_gather(tf, [vf + (16 + iota)])
            for c in range(4):
                am[i, pl.ds(64 + c * 16, 16)] = plsc.load_gather(
                    rr, [row, roff + (c * 16 + iota)])
            for c in range(4):
                am[i, pl.ds(128 + c * 16, 16)] = plsc.load_gather(
                    rp, [row, poff + (c * 16 + iota)])
            return carry3

        bb = row_b0 + j

        def emit(s_const):
            rc, rr, rp = dsts[s_const]
            lax.fori_loop(
                0, _C,
                lambda i, c: assemble(i, c, rr, rp, asm[s_const]), 0)
            pltpu.make_async_copy(
                rc, out.at[bb, :, pl.ds(0, 128)], semw[s_const]).start()
            pltpu.make_async_copy(
                asm[s_const], out.at[bb, :, pl.ds(128, 192)],
                semw[s_const]).start()

        @pl.when(s == 0)
        def _():
            emit(0)

        @pl.when(s == 1)
        def _():
            emit(1)

        return carry

    lax.fori_loop(0, _STEPS, step, 0)
    wait_writes((_STEPS - 1) % 2)


def kernel(idx_char, idx_bound, idx_flag, idx_radical, idx_pinyin,
           W_char, W_bound, W_flag, W_radical, W_pinyin):
    idxs = [a.reshape(_N).astype(jnp.int32)
            for a in (idx_char, idx_bound, idx_flag, idx_radical, idx_pinyin)]
    tables = [W_char,
              W_bound.reshape(100 * 32),
              W_flag.reshape(100 * 32),
              W_radical.reshape(500, 128),
              W_pinyin.reshape(500, 128)]
    scratch = ([pltpu.VMEM((_PER_W,), jnp.int32) for _ in range(5)]
               + [pltpu.VMEM((100 * 32,), jnp.float32) for _ in range(2)]
               + [pltpu.VMEM((_C,), jnp.int32) for _ in range(6)]
               + [pltpu.VMEM((_C, 128), jnp.float32) for _ in range(6)]
               + [pltpu.VMEM((_C, 192), jnp.float32) for _ in range(2)]
               + [pltpu.SemaphoreType.DMA for _ in range(4)])
    halves = []
    for half in (0, 1):
        body = functools.partial(_sc_body, half)
        k = pl.kernel(
            body,
            out_type=jax.ShapeDtypeStruct((_BH, _L, _TOT), jnp.float32),
            mesh=plsc.VectorSubcoreMesh(core_axis_name="c",
                                        subcore_axis_name="s"),
            scratch_types=scratch,
            compiler_params=pltpu.CompilerParams(needs_layout_passes=False),
        )
        halves.append(k(*idxs, *tables))
    return jnp.concatenate(halves, axis=0)


# speedup vs baseline: 2.3519x; 1.2000x over previous
import jax
import jax.numpy as jnp
from jax import lax
from jax.experimental import pallas as pl
from jax.experimental.pallas import tpu as pltpu
from jax.experimental.pallas import tpu_sc as plsc

_TOT = 320
_B, _L = 4096, 50
_N = _B * _L
_C = 50
_NC, _NS = 2, 16
_NW = _NC * _NS
_PER_W = _N // _NW
_STEPS = _PER_W // _C


def _sc_body(ic, ib, if_, ir, ip, wc, wb, wf, wr, wp, out,
             ivc, ivb, ivf, ivr, ivp, tb, tf,
             q0c, q0r, q0p, q1c, q1r, q1p,
             c0, c1, r0, r1, p0, p1, a0, a1,
             sg0, sg1, sw0, sw1):
    wid = lax.axis_index("s") * _NC + lax.axis_index("c")
    tok0 = wid * _PER_W
    row_b0 = wid * _STEPS
    gsrc = (wc, wr, wp)
    qidx = ((q0c, q0r, q0p), (q1c, q1r, q1p))
    dsts = ((c0, r0, p0), (c1, r1, p1))
    asm = (a0, a1)
    semg = (sg0, sg1)
    semw = (sw0, sw1)

    for src, dst in ((ic, ivc), (ib, ivb), (if_, ivf), (ir, ivr), (ip, ivp)):
        pltpu.sync_copy(src.at[pl.ds(tok0, _PER_W)], dst)
    pltpu.sync_copy(wb, tb)
    pltpu.sync_copy(wf, tf)

    iota = lax.iota(jnp.int32, 16)
    tail_mask = iota < 2

    def repack(j, s):
        base = jnp.full((16,), j * _C, jnp.int32)
        for t, (flat, shift) in enumerate(((ivc, 0), (ivr, 1), (ivp, 1))):
            q = qidx[s][t]
            for c in range(3):
                v = plsc.load_gather(flat, [base + (c * 16 + iota)])
                q[pl.ds(c * 16, 16)] = lax.shift_right_logical(v, shift)
            v = plsc.load_gather(flat, [base + (48 + iota)])
            plsc.store_scatter(q, [48 + iota],
                               lax.shift_right_logical(v, shift),
                               mask=tail_mask)

    def fire_gathers(s):
        for t in range(3):
            pltpu.make_async_copy(
                gsrc[t].at[qidx[s][t]], dsts[s][t], semg[s]).start()

    def wait_gathers(s):
        for t in range(3):
            pltpu.make_async_copy(
                out.at[0, :, pl.ds(0, 128)], dsts[s][t], semg[s]).wait()

    def wait_writes(s):
        pltpu.make_async_copy(
            dsts[s][0], out.at[0, :, pl.ds(0, 128)], semw[s]).wait()
        pltpu.make_async_copy(
            asm[s], out.at[0, :, pl.ds(128, 192)], semw[s]).wait()

    repack(0, 0)
    fire_gathers(0)

    def step(j, carry):
        s = lax.rem(j, 2)

        @pl.when(s == 0)
        def _():
            wait_gathers(0)

        @pl.when(s == 1)
        def _():
            wait_gathers(1)

        @pl.when((j >= 1) & (s == 0))
        def _():
            wait_writes(1)

        @pl.when((j >= 1) & (s == 1))
        def _():
            wait_writes(0)

        @pl.when((j < _STEPS - 1) & (s == 0))
        def _():
            repack(j + 1, 1)
            fire_gathers(1)

        @pl.when((j < _STEPS - 1) & (s == 1))
        def _():
            repack(j + 1, 0)
            fire_gathers(0)

        def assemble(i, carry3, rr, rp, am):
            pos = jnp.full((16,), j * _C + i, jnp.int32)
            row = jnp.full((16,), i, jnp.int32)
            vb = plsc.load_gather(ivb, [pos]) * 32
            vf = plsc.load_gather(ivf, [pos]) * 32
            vr = plsc.load_gather(ivr, [pos])
            vp = plsc.load_gather(ivp, [pos])
            roff = (vr & 1) * 64
            poff = (vp & 1) * 64
            am[i, pl.ds(0, 16)] = plsc.load_gather(tb, [vb + iota])
            am[i, pl.ds(16, 16)] = plsc.load_gather(tb, [vb + (16 + iota)])
            am[i, pl.ds(32, 16)] = plsc.load_gather(tf, [vf + iota])
            am[i, pl.ds(48, 16)] = plsc.load_gather(tf, [vf + (16 + iota)])
            for c in range(4):
                am[i, pl.ds(64 + c * 16, 16)] = plsc.load_gather(
                    rr, [row, roff + (c * 16 + iota)])
            for c in range(4):
                am[i, pl.ds(128 + c * 16, 16)] = plsc.load_gather(
                    rp, [row, poff + (c * 16 + iota)])
            return carry3

        bb = row_b0 + j

        def emit(s_const):
            rc, rr, rp = dsts[s_const]
            lax.fori_loop(
                0, _C,
                lambda i, c: assemble(i, c, rr, rp, asm[s_const]), 0)
            pltpu.make_async_copy(
                rc, out.at[bb, :, pl.ds(0, 128)], semw[s_const]).start()
            pltpu.make_async_copy(
                asm[s_const], out.at[bb, :, pl.ds(128, 192)],
                semw[s_const]).start()

        @pl.when(s == 0)
        def _():
            emit(0)

        @pl.when(s == 1)
        def _():
            emit(1)

        return carry

    lax.fori_loop(0, _STEPS, step, 0)
    wait_writes((_STEPS - 1) % 2)


def kernel(idx_char, idx_bound, idx_flag, idx_radical, idx_pinyin,
           W_char, W_bound, W_flag, W_radical, W_pinyin):
    idxs = [a.reshape(_N).astype(jnp.int32)
            for a in (idx_char, idx_bound, idx_flag, idx_radical, idx_pinyin)]
    tables = [W_char,
              W_bound.reshape(100 * 32),
              W_flag.reshape(100 * 32),
              W_radical.reshape(500, 128),
              W_pinyin.reshape(500, 128)]
    scratch = ([pltpu.VMEM((_PER_W,), jnp.int32) for _ in range(5)]
               + [pltpu.VMEM((100 * 32,), jnp.float32) for _ in range(2)]
               + [pltpu.VMEM((_C,), jnp.int32) for _ in range(6)]
               + [pltpu.VMEM((_C, 128), jnp.float32) for _ in range(6)]
               + [pltpu.VMEM((_C, 192), jnp.float32) for _ in range(2)]
               + [pltpu.SemaphoreType.DMA for _ in range(4)])
    k = pl.kernel(
        _sc_body,
        out_type=jax.ShapeDtypeStruct((_B, _L, _TOT), jnp.float32),
        mesh=plsc.VectorSubcoreMesh(core_axis_name="c", subcore_axis_name="s"),
        scratch_types=scratch,
        compiler_params=pltpu.CompilerParams(needs_layout_passes=False),
    )
    return k(*idxs, *tables)
